# baseline (speedup 1.0000x reference)
import jax
import jax.numpy as jnp
from jax.experimental import pallas as pl
from jax.experimental.pallas import tpu as pltpu

B, S, D = 8, 512, 768
C = 5
H1 = 768
H2 = 384
CP = 128


def _mlp_body(nseg_ref, q_ref, seg_ref, tis_ref, w1q_ref, w1s_ref, b1_ref,
              w2_ref, b2_ref, w3_ref, b3_ref, probs_ref, tinst_ref):
    b = pl.program_id(0)
    n = nseg_ref[b]

    row = jax.lax.broadcasted_iota(jnp.int32, (S, 1), 0)
    mask = row < n

    x = jnp.where(mask, seg_ref[0], 0.0)

    qh = jnp.dot(q_ref[0], w1q_ref[...],
                 preferred_element_type=jnp.float32,
                 precision=jax.lax.Precision.HIGHEST)
    h1 = jnp.dot(x, w1s_ref[...],
                 preferred_element_type=jnp.float32,
                 precision=jax.lax.Precision.HIGHEST)
    h1 = jax.nn.gelu(h1 + qh + b1_ref[...])
    h2 = jnp.dot(h1, w2_ref[...],
                 preferred_element_type=jnp.float32,
                 precision=jax.lax.Precision.HIGHEST)
    h2 = jax.nn.gelu(h2 + b2_ref[...])
    logits = jnp.dot(h2, w3_ref[...],
                     preferred_element_type=jnp.float32,
                     precision=jax.lax.Precision.HIGHEST)
    logits = logits + b3_ref[...]

    lane = jax.lax.broadcasted_iota(jnp.int32, (S, CP), 1)
    logits = jnp.where(lane < C, logits, -1e30)
    m = jnp.max(logits, axis=-1, keepdims=True)
    e = jnp.exp(logits - m)
    probs_ref[0] = e / jnp.sum(e, axis=-1, keepdims=True)

    col = jax.lax.broadcasted_iota(jnp.int32, (1, S), 1)
    tinst_ref[0] = jnp.where(col < n, tis_ref[0], 1.0)


def kernel(questions_embedding, context_segments_embedding,
           num_context_segments, target_agg_score, target_instance_score,
           W1, b1, W2, b2, W3, b3):
    w1q = W1[:D]
    w1s = W1[D:]
    b1_2d = b1.reshape(1, H1)
    b2_2d = b2.reshape(1, H2)
    w3p = jnp.pad(W3, ((0, 0), (0, CP - C)))
    b3p = jnp.pad(b3, (0, CP - C)).reshape(1, CP)

    grid_spec = pltpu.PrefetchScalarGridSpec(
        num_scalar_prefetch=1,
        grid=(B,),
        in_specs=[
            pl.BlockSpec((1, 1, D), lambda b, n: (b, 0, 0)),
            pl.BlockSpec((1, S, D), lambda b, n: (b, 0, 0)),
            pl.BlockSpec((1, 1, S), lambda b, n: (b, 0, 0)),
            pl.BlockSpec((D, H1), lambda b, n: (0, 0)),
            pl.BlockSpec((D, H1), lambda b, n: (0, 0)),
            pl.BlockSpec((1, H1), lambda b, n: (0, 0)),
            pl.BlockSpec((H1, H2), lambda b, n: (0, 0)),
            pl.BlockSpec((1, H2), lambda b, n: (0, 0)),
            pl.BlockSpec((H2, CP), lambda b, n: (0, 0)),
            pl.BlockSpec((1, CP), lambda b, n: (0, 0)),
        ],
        out_specs=[
            pl.BlockSpec((1, S, CP), lambda b, n: (b, 0, 0)),
            pl.BlockSpec((1, 1, S), lambda b, n: (b, 0, 0)),
        ],
    )

    probs_p, tinst = pl.pallas_call(
        _mlp_body,
        grid_spec=grid_spec,
        out_shape=[
            jax.ShapeDtypeStruct((B, S, CP), jnp.float32),
            jax.ShapeDtypeStruct((B, 1, S), jnp.float32),
        ],
    )(num_context_segments, questions_embedding.reshape(B, 1, D),
      context_segments_embedding,
      target_instance_score.reshape(B, 1, S), w1q, w1s, b1_2d, W2, b2_2d,
      w3p, b3p)

    probs = probs_p[:, :, :C]
    return (target_agg_score, tinst.reshape(B, S), probs,
            num_context_segments)


# speedup vs baseline: 2.7024x; 2.7024x over previous
import jax
import jax.numpy as jnp
from jax.experimental import pallas as pl
from jax.experimental.pallas import tpu as pltpu

B, S, D = 8, 512, 768
C = 5
H1 = 768
H2 = 384
CP = 128


def _mlp_body(nseg_ref, q_ref, seg_ref, tis_ref, w1q_ref, w1s_ref, b1_ref,
              w2_ref, b2_ref, w3_ref, b3_ref, probs_ref, tinst_ref):
    b = pl.program_id(0)
    n = nseg_ref[b]

    row = jax.lax.broadcasted_iota(jnp.int32, (S, 1), 0)
    mask = row < n

    x = jnp.where(mask, seg_ref[0], 0.0)

    qh = jnp.dot(q_ref[0], w1q_ref[...],
                 preferred_element_type=jnp.float32,
                 precision=jax.lax.Precision.DEFAULT)
    h1 = jnp.dot(x, w1s_ref[...],
                 preferred_element_type=jnp.float32,
                 precision=jax.lax.Precision.DEFAULT)
    h1 = jax.nn.gelu(h1 + qh + b1_ref[...])
    h2 = jnp.dot(h1, w2_ref[...],
                 preferred_element_type=jnp.float32,
                 precision=jax.lax.Precision.DEFAULT)
    h2 = jax.nn.gelu(h2 + b2_ref[...])
    logits = jnp.dot(h2, w3_ref[...],
                     preferred_element_type=jnp.float32,
                     precision=jax.lax.Precision.DEFAULT)
    logits = logits + b3_ref[...]

    lane = jax.lax.broadcasted_iota(jnp.int32, (S, CP), 1)
    logits = jnp.where(lane < C, logits, -1e30)
    m = jnp.max(logits, axis=-1, keepdims=True)
    e = jnp.exp(logits - m)
    probs_ref[0] = e / jnp.sum(e, axis=-1, keepdims=True)

    col = jax.lax.broadcasted_iota(jnp.int32, (1, S), 1)
    tinst_ref[0] = jnp.where(col < n, tis_ref[0], 1.0)


def kernel(questions_embedding, context_segments_embedding,
           num_context_segments, target_agg_score, target_instance_score,
           W1, b1, W2, b2, W3, b3):
    w1q = W1[:D]
    w1s = W1[D:]
    b1_2d = b1.reshape(1, H1)
    b2_2d = b2.reshape(1, H2)
    w3p = jnp.pad(W3, ((0, 0), (0, CP - C)))
    b3p = jnp.pad(b3, (0, CP - C)).reshape(1, CP)

    grid_spec = pltpu.PrefetchScalarGridSpec(
        num_scalar_prefetch=1,
        grid=(B,),
        in_specs=[
            pl.BlockSpec((1, 1, D), lambda b, n: (b, 0, 0)),
            pl.BlockSpec((1, S, D), lambda b, n: (b, 0, 0)),
            pl.BlockSpec((1, 1, S), lambda b, n: (b, 0, 0)),
            pl.BlockSpec((D, H1), lambda b, n: (0, 0)),
            pl.BlockSpec((D, H1), lambda b, n: (0, 0)),
            pl.BlockSpec((1, H1), lambda b, n: (0, 0)),
            pl.BlockSpec((H1, H2), lambda b, n: (0, 0)),
            pl.BlockSpec((1, H2), lambda b, n: (0, 0)),
            pl.BlockSpec((H2, CP), lambda b, n: (0, 0)),
            pl.BlockSpec((1, CP), lambda b, n: (0, 0)),
        ],
        out_specs=[
            pl.BlockSpec((1, S, CP), lambda b, n: (b, 0, 0)),
            pl.BlockSpec((1, 1, S), lambda b, n: (b, 0, 0)),
        ],
    )

    probs_p, tinst = pl.pallas_call(
        _mlp_body,
        grid_spec=grid_spec,
        out_shape=[
            jax.ShapeDtypeStruct((B, S, CP), jnp.float32),
            jax.ShapeDtypeStruct((B, 1, S), jnp.float32),
        ],
    )(num_context_segments, questions_embedding.reshape(B, 1, D),
      context_segments_embedding,
      target_instance_score.reshape(B, 1, S), w1q, w1s, b1_2d, W2, b2_2d,
      w3p, b3p)

    probs = probs_p[:, :, :C]
    return (target_agg_score, tinst.reshape(B, S), probs,
            num_context_segments)
